# baseline (speedup 1.0000x reference)
import functools

import jax
import jax.numpy as jnp
from jax import lax
from jax.experimental import pallas as pl
from jax.experimental.pallas import tpu as pltpu
from jax.experimental.pallas import tpu_sc as plsc

NC = 2
NS = 16
L = 16
NW = NC * NS
D = 64
CH = 128


def _body(pk_hbm, ent_hbm, rel_hbm, out_hbm,
          idx_v0, idx_v1, eh_v0, eh_v1, et_v0, et_v1, rl_v0, rl_v1,
          out_v, sem0, sem1, *, n_chunks):
    wid = lax.axis_index("s") * NC + lax.axis_index("c")
    rows0 = jnp.arange(L, dtype=jnp.int32)
    idx_bufs = (idx_v0, idx_v1)
    eh_bufs = (eh_v0, eh_v1)
    et_bufs = (et_v0, et_v1)
    rl_bufs = (rl_v0, rl_v1)
    sems = (sem0, sem1)

    def fire(c, b):
        pltpu.sync_copy(pk_hbm.at[wid * n_chunks + c], idx_bufs[b])
        pltpu.async_copy(ent_hbm.at[idx_bufs[b].at[0]], eh_bufs[b], sems[b])
        pltpu.async_copy(ent_hbm.at[idx_bufs[b].at[1]], et_bufs[b], sems[b])
        pltpu.async_copy(rel_hbm.at[idx_bufs[b].at[2]], rl_bufs[b], sems[b])

    def drain(b):
        pltpu.make_async_copy(ent_hbm.at[pl.ds(0, CH)], eh_bufs[b], sems[b]).wait()
        pltpu.make_async_copy(ent_hbm.at[pl.ds(0, CH)], et_bufs[b], sems[b]).wait()
        pltpu.make_async_copy(rel_hbm.at[pl.ds(0, CH)], rl_bufs[b], sems[b]).wait()

    fire(0, 0)
    for c in range(n_chunks):
        b = c % 2
        if c + 1 < n_chunks:
            fire(c + 1, 1 - b)
        drain(b)
        eh_v, et_v, rl_v = eh_bufs[b], et_bufs[b], rl_bufs[b]

        def group_body(g, _):
            rows = g * L + rows0

            def d_body(d, acc):
                cols = jnp.full((L,), d, dtype=jnp.int32)
                cols_im = cols + D
                ehre = plsc.load_gather(eh_v, [rows, cols])
                ehim = plsc.load_gather(eh_v, [rows, cols_im])
                etre = plsc.load_gather(et_v, [rows, cols])
                etim = plsc.load_gather(et_v, [rows, cols_im])
                rre = plsc.load_gather(rl_v, [rows, cols])
                rim = plsc.load_gather(rl_v, [rows, cols_im])
                return (acc + rre * (ehre * etre + ehim * etim)
                        + rim * (ehre * etim - ehim * etre))

            acc = lax.fori_loop(0, D, d_body, jnp.zeros((L,), jnp.float32))
            out_v[pl.ds(g * L, L)] = acc
            return 0

        lax.fori_loop(0, CH // L, group_body, 0)
        pltpu.sync_copy(out_v, out_hbm.at[pl.ds((wid * n_chunks + c) * CH, CH)])


def kernel(hs, rs, ts, ent_re, ent_im, rel_re, rel_im):
    batch = hs.shape[0]
    n_chunks = batch // NW // CH
    ent = jnp.concatenate([ent_re, ent_im], axis=1)
    rel = jnp.concatenate([rel_re, rel_im], axis=1)
    pk = jnp.stack([hs, ts, rs], axis=0)
    pk = pk.reshape(3, batch // CH, CH).transpose(1, 0, 2)
    mesh = plsc.VectorSubcoreMesh(core_axis_name="c", subcore_axis_name="s")
    k = pl.kernel(
        functools.partial(_body, n_chunks=n_chunks),
        out_type=jax.ShapeDtypeStruct((batch,), jnp.float32),
        mesh=mesh,
        compiler_params=pltpu.CompilerParams(needs_layout_passes=False),
        scratch_types=[
            pltpu.VMEM((3, CH), jnp.int32),
            pltpu.VMEM((3, CH), jnp.int32),
            pltpu.VMEM((CH, 2 * D), jnp.float32),
            pltpu.VMEM((CH, 2 * D), jnp.float32),
            pltpu.VMEM((CH, 2 * D), jnp.float32),
            pltpu.VMEM((CH, 2 * D), jnp.float32),
            pltpu.VMEM((CH, 2 * D), jnp.float32),
            pltpu.VMEM((CH, 2 * D), jnp.float32),
            pltpu.VMEM((CH,), jnp.float32),
            pltpu.SemaphoreType.DMA,
            pltpu.SemaphoreType.DMA,
        ],
    )
    return k(pk, ent, rel)
